# baseline (speedup 1.0000x reference)
import jax
import jax.numpy as jnp
from jax.experimental import pallas as pl
from jax.experimental.pallas import tpu as pltpu

_NUM_POINT = 128
_INIT_STRIDE = 10.0
_COARSE_STRIDE = 4.0
_DOWN_SAMPLE = 4.0
_TH = 8


def _conv_body(x_ref, w1_ref, b1_ref, w2_ref, b2_ref, out_ref):
    xb = x_ref[0]
    w1 = w1_ref[...]
    w2 = w2_ref[...]
    b1 = b1_ref[...]
    b2 = b2_ref[...]
    for h0 in range(0, 128, _TH):
        pieces = []
        for dy in range(3):
            for dx in range(3):
                xs = xb[h0 + dy:h0 + dy + _TH, dx:dx + 128, :]
                pieces.append(xs.reshape(_TH * 128, 64))
        x9 = jnp.concatenate(pieces, axis=1)
        acc = jnp.dot(x9, w1, preferred_element_type=jnp.float32) + b1
        acc = jnp.maximum(acc, 0.0)
        z = jnp.dot(acc, w2, preferred_element_type=jnp.float32) + b2
        out_ref[0, h0:h0 + _TH, :, :] = z.reshape(_TH, 128, 64)


def _fused_conv(x_nhwc_pad, conv1_w, conv1_b, conv2_w, conv2_b):
    B = x_nhwc_pad.shape[0]
    w1 = conv1_w.transpose(2, 3, 1, 0).reshape(576, 256)
    w2 = conv2_w[:, :, 0, 0].T
    return pl.pallas_call(
        _conv_body,
        grid=(B,),
        in_specs=[
            pl.BlockSpec((1, 130, 130, 64), lambda b: (b, 0, 0, 0)),
            pl.BlockSpec((576, 256), lambda b: (0, 0)),
            pl.BlockSpec((1, 256), lambda b: (0, 0)),
            pl.BlockSpec((256, 64), lambda b: (0, 0)),
            pl.BlockSpec((1, 64), lambda b: (0, 0)),
        ],
        out_specs=pl.BlockSpec((1, 128, 128, 64), lambda b: (b, 0, 0, 0)),
        out_shape=jax.ShapeDtypeStruct((B, 128, 128, 64), jnp.float32),
    )(x_nhwc_pad, w1, conv1_b.reshape(1, 256), w2, conv2_b.reshape(1, 64))


def _tail_body(fp_ref, ip_ref, pw_ref, fw_ref, fb_ref, pi_ref, pc_ref):
    fp = fp_ref[...]
    t = jax.lax.dot_general(fp, pw_ref[...], (((1,), (1,)), ((), ())),
                            preferred_element_type=jnp.float32)
    offs = jax.lax.dot_general(t, fw_ref[...], (((1,), (1,)), ((), ())),
                               preferred_element_type=jnp.float32)
    offs = offs + fb_ref[...]
    ip = ip_ref[...]
    pi_ref[...] = ip * _DOWN_SAMPLE
    pc_ref[...] = offs * (_COARSE_STRIDE * _DOWN_SAMPLE) + ip * _DOWN_SAMPLE


def kernel(ct_hm, wh, cnn_feature, ct_01, ct_ind, ct_img_idx,
           conv1_w, conv1_b, conv2_w, conv2_b, poly_w, fuse_w, fuse_b):
    B, _, H, W = ct_hm.shape
    mask = ct_01.reshape(-1)
    ind = jnp.where(mask, ct_ind.reshape(-1), 0).astype(jnp.int32)
    img = jnp.where(mask, ct_img_idx.reshape(-1), 0).astype(jnp.int32)
    N = mask.shape[0]
    ct_x = ind % W
    ct_y = ind // W
    ct_offset = wh[img, :, ct_y, ct_x].reshape(N, -1, 2)
    ct = jnp.stack([ct_x.astype(jnp.float32), ct_y.astype(jnp.float32)], axis=1)
    init_polys = ct_offset * _INIT_STRIDE + ct[:, None, :]

    x_nhwc = jnp.transpose(cnn_feature, (0, 2, 3, 1))
    x_pad = jnp.pad(x_nhwc, ((0, 0), (1, 1), (1, 1), (0, 0)))
    feat = _fused_conv(x_pad, conv1_w, conv1_b, conv2_w, conv2_b)

    points = jnp.concatenate([ct[:, None, :], init_polys], axis=1)
    x = points[..., 0] - 0.5
    y = points[..., 1] - 0.5
    x0 = jnp.floor(x)
    y0 = jnp.floor(y)

    def gather(yc, xc):
        valid = (xc >= 0) & (xc < W) & (yc >= 0) & (yc < H)
        xi = jnp.clip(xc, 0, W - 1).astype(jnp.int32)
        yi = jnp.clip(yc, 0, H - 1).astype(jnp.int32)
        v = feat[img[:, None], yi, xi, :]
        return v * valid[..., None].astype(feat.dtype)

    wx1 = x - x0
    wx0 = 1.0 - wx1
    wy1 = y - y0
    wy0 = 1.0 - wy1
    out = (gather(y0, x0) * (wy0 * wx0)[..., None]
           + gather(y0, x0 + 1) * (wy0 * wx1)[..., None]
           + gather(y0 + 1, x0) * (wy1 * wx0)[..., None]
           + gather(y0 + 1, x0 + 1) * (wy1 * wx1)[..., None])
    fp = out.reshape(N, -1)

    pw_perm = poly_w.reshape(512, 64, 129).transpose(0, 2, 1).reshape(512, 8256)

    ip_flat = init_polys.reshape(N, _NUM_POINT * 2)
    pi, pc = pl.pallas_call(
        _tail_body,
        out_shape=(jax.ShapeDtypeStruct((N, _NUM_POINT * 2), jnp.float32),
                   jax.ShapeDtypeStruct((N, _NUM_POINT * 2), jnp.float32)),
    )(fp, ip_flat, pw_perm, fuse_w, fuse_b.reshape(1, -1))
    return (pi.reshape(N, _NUM_POINT, 2), pc.reshape(N, _NUM_POINT, 2))


# speedup vs baseline: 1.0942x; 1.0942x over previous
import jax
import jax.numpy as jnp
from jax.experimental import pallas as pl
from jax.experimental.pallas import tpu as pltpu

_NUM_POINT = 128
_INIT_STRIDE = 10.0
_COARSE_STRIDE = 4.0
_DOWN_SAMPLE = 4.0
_TH = 8


def _conv_body(x_ref, w1_ref, b1_ref, w2_ref, b2_ref, out_ref, xt_s):
    w1 = w1_ref[...]
    w2 = w2_ref[...]
    b1 = b1_ref[...]
    b2 = b2_ref[...]
    xt_s[0:1, :, :] = jnp.zeros((1, 130, 64), jnp.float32)
    xt_s[129:130, :, :] = jnp.zeros((1, 130, 64), jnp.float32)
    xt_s[1:129, 0:1, :] = jnp.zeros((128, 1, 64), jnp.float32)
    xt_s[1:129, 129:130, :] = jnp.zeros((128, 1, 64), jnp.float32)
    for r0 in range(0, 128, 16):
        blk = x_ref[0][:, r0:r0 + 16, :].reshape(64, 16 * 128)
        xt_s[r0 + 1:r0 + 17, 1:129, :] = blk.T.reshape(16, 128, 64)
    for h0 in range(0, 128, _TH):
        pieces = []
        for dy in range(3):
            for dx in range(3):
                xs = xt_s[h0 + dy:h0 + dy + _TH, dx:dx + 128, :]
                pieces.append(xs.reshape(_TH * 128, 64))
        x9 = jnp.concatenate(pieces, axis=1)
        acc = jnp.dot(x9, w1, preferred_element_type=jnp.float32) + b1
        acc = jnp.maximum(acc, 0.0)
        z = jnp.dot(acc, w2, preferred_element_type=jnp.float32) + b2
        out_ref[0, h0:h0 + _TH, :, :] = z.reshape(_TH, 128, 64)


def _fused_conv(x_nchw, conv1_w, conv1_b, conv2_w, conv2_b):
    B = x_nchw.shape[0]
    w1 = conv1_w.transpose(2, 3, 1, 0).reshape(576, 256)
    w2 = conv2_w[:, :, 0, 0].T
    return pl.pallas_call(
        _conv_body,
        grid=(B,),
        in_specs=[
            pl.BlockSpec((1, 64, 128, 128), lambda b: (b, 0, 0, 0)),
            pl.BlockSpec((576, 256), lambda b: (0, 0)),
            pl.BlockSpec((1, 256), lambda b: (0, 0)),
            pl.BlockSpec((256, 64), lambda b: (0, 0)),
            pl.BlockSpec((1, 64), lambda b: (0, 0)),
        ],
        out_specs=pl.BlockSpec((1, 128, 128, 64), lambda b: (b, 0, 0, 0)),
        out_shape=jax.ShapeDtypeStruct((B, 128, 128, 64), jnp.float32),
        scratch_shapes=[pltpu.VMEM((130, 130, 64), jnp.float32)],
    )(x_nchw, w1, conv1_b.reshape(1, 256), w2, conv2_b.reshape(1, 64))


def _tail_body(fp_ref, ip_ref, pw_ref, fw_ref, fb_ref, pi_ref, pc_ref):
    fp = fp_ref[...]
    t = jax.lax.dot_general(fp, pw_ref[...], (((1,), (1,)), ((), ())),
                            preferred_element_type=jnp.float32)
    offs = jax.lax.dot_general(t, fw_ref[...], (((1,), (1,)), ((), ())),
                               preferred_element_type=jnp.float32)
    offs = offs + fb_ref[...]
    ip = ip_ref[...]
    pi_ref[...] = ip * _DOWN_SAMPLE
    pc_ref[...] = offs * (_COARSE_STRIDE * _DOWN_SAMPLE) + ip * _DOWN_SAMPLE


def kernel(ct_hm, wh, cnn_feature, ct_01, ct_ind, ct_img_idx,
           conv1_w, conv1_b, conv2_w, conv2_b, poly_w, fuse_w, fuse_b):
    B, _, H, W = ct_hm.shape
    mask = ct_01.reshape(-1)
    ind = jnp.where(mask, ct_ind.reshape(-1), 0).astype(jnp.int32)
    img = jnp.where(mask, ct_img_idx.reshape(-1), 0).astype(jnp.int32)
    N = mask.shape[0]
    ct_x = ind % W
    ct_y = ind // W
    ct_offset = wh[img, :, ct_y, ct_x].reshape(N, -1, 2)
    ct = jnp.stack([ct_x.astype(jnp.float32), ct_y.astype(jnp.float32)], axis=1)
    init_polys = ct_offset * _INIT_STRIDE + ct[:, None, :]

    feat = _fused_conv(cnn_feature, conv1_w, conv1_b, conv2_w, conv2_b)

    points = jnp.concatenate([ct[:, None, :], init_polys], axis=1)
    x = points[..., 0] - 0.5
    y = points[..., 1] - 0.5
    x0 = jnp.floor(x)
    y0 = jnp.floor(y)

    def gather(yc, xc):
        valid = (xc >= 0) & (xc < W) & (yc >= 0) & (yc < H)
        xi = jnp.clip(xc, 0, W - 1).astype(jnp.int32)
        yi = jnp.clip(yc, 0, H - 1).astype(jnp.int32)
        v = feat[img[:, None], yi, xi, :]
        return v * valid[..., None].astype(feat.dtype)

    wx1 = x - x0
    wx0 = 1.0 - wx1
    wy1 = y - y0
    wy0 = 1.0 - wy1
    out = (gather(y0, x0) * (wy0 * wx0)[..., None]
           + gather(y0, x0 + 1) * (wy0 * wx1)[..., None]
           + gather(y0 + 1, x0) * (wy1 * wx0)[..., None]
           + gather(y0 + 1, x0 + 1) * (wy1 * wx1)[..., None])
    fp = out.reshape(N, -1)

    pw_perm = poly_w.reshape(512, 64, 129).transpose(0, 2, 1).reshape(512, 8256)

    ip_flat = init_polys.reshape(N, _NUM_POINT * 2)
    pi, pc = pl.pallas_call(
        _tail_body,
        out_shape=(jax.ShapeDtypeStruct((N, _NUM_POINT * 2), jnp.float32),
                   jax.ShapeDtypeStruct((N, _NUM_POINT * 2), jnp.float32)),
    )(fp, ip_flat, pw_perm, fuse_w, fuse_b.reshape(1, -1))
    return (pi.reshape(N, _NUM_POINT, 2), pc.reshape(N, _NUM_POINT, 2))
